# baseline (speedup 1.0000x reference)
import functools

import jax
import jax.numpy as jnp
from jax.experimental import pallas as pl
from jax.experimental.pallas import tpu as pltpu

NUM_GATES = 8
CAPACITY_FACTOR = 1.25
MIN_EXPERT_CAPACITY = 4
TBLK = 1024


def _route_kernel(eb_ref, out_ref, *, cap, k_total):
    b = pl.program_id(0)
    tb = pl.program_id(1)
    e = eb_ref[b]
    t0 = tb * TBLK
    t = jax.lax.broadcasted_iota(jnp.int32, (TBLK, k_total), 0) + t0
    k = jax.lax.broadcasted_iota(jnp.int32, (TBLK, k_total), 1)
    val = jnp.where((t < cap) & (k == e * cap + t), 1.0, 0.0).astype(jnp.float32)
    out_ref[0] = val


def kernel(x, current_y):
    b, gs, _ = x.shape
    cap = int(gs * CAPACITY_FACTOR / NUM_GATES)
    cap = max(min(gs, cap), MIN_EXPERT_CAPACITY)
    k_total = NUM_GATES * cap

    eb = jnp.remainder(current_y.astype(jnp.int32), NUM_GATES)

    kern = functools.partial(_route_kernel, cap=cap, k_total=k_total)
    grid_spec = pltpu.PrefetchScalarGridSpec(
        num_scalar_prefetch=1,
        grid=(b, gs // TBLK),
        in_specs=[],
        out_specs=[
            pl.BlockSpec((1, TBLK, k_total), lambda i, j, eb_ref: (i, j, 0)),
        ],
    )
    out_shape = [
        jax.ShapeDtypeStruct((b, gs, k_total), jnp.float32),
    ]
    (out,) = pl.pallas_call(
        kern, grid_spec=grid_spec, out_shape=out_shape,
        compiler_params=pltpu.CompilerParams(
            dimension_semantics=("parallel", "parallel")),
    )(eb)
    dispatch = out.reshape(b, gs, NUM_GATES, cap)
    combine = jnp.abs(out).reshape(b, gs, NUM_GATES, cap)
    return dispatch, combine
